# baseline (speedup 1.0000x reference)
import jax
import jax.numpy as jnp
from jax import lax
from jax.experimental import pallas as pl
from jax.experimental.pallas import tpu as pltpu
import numpy as np

B = 64
S = 8
V = 100000
VB = 12800
NBLK = (V + VB - 1) // VB

_KS0 = np.uint32(0)
_KS1 = np.uint32(42)
_KS2 = np.uint32(42) ^ np.uint32(0x1BD11BDA)
_R0 = (13, 15, 26, 6)
_R1 = (17, 29, 16, 24)


def _threefry_bits(cnt):
    x0 = jnp.zeros_like(cnt)
    x1 = cnt + _KS1

    def rnd(x0, x1, r):
        x0 = x0 + x1
        x1 = (x1 << np.uint32(r)) | (x1 >> np.uint32(32 - r))
        return x0, x1 ^ x0

    sched = ((_R0, _KS1, _KS2, 1), (_R1, _KS2, _KS0, 2), (_R0, _KS0, _KS1, 3),
             (_R1, _KS1, _KS2, 4), (_R0, _KS2, _KS0, 5))
    for rots, a0, a1, c in sched:
        for r in rots:
            x0, x1 = rnd(x0, x1, r)
        x0 = x0 + a0
        x1 = x1 + (a1 + np.uint32(c))
    return x0 ^ x1


def _gumbel_from_bits(bits):
    fb = (bits >> np.uint32(9)) | np.uint32(0x3F800000)
    u = lax.bitcast_convert_type(fb, jnp.float32) - jnp.float32(1.0)
    tiny = jnp.float32(np.finfo(np.float32).tiny)
    u = jnp.maximum(tiny, u * (jnp.float32(1.0) - tiny) + tiny)
    return -jnp.log(-jnp.log(u))


def _compose_kernel(x_ref, mask_ref, final_ref, ids_ref, m_scr, i_scr):
    j = pl.program_id(0)

    x = x_ref[...]
    final = x + mask_ref[0, :][None, :]
    final_ref[...] = final

    col = jax.lax.broadcasted_iota(jnp.int32, (B, VB), 1) + j * VB
    row = jax.lax.broadcasted_iota(jnp.int32, (B, VB), 0)
    cnt = (row * V + col).astype(jnp.uint32)
    g = _gumbel_from_bits(_threefry_bits(cnt))

    y = jnp.where(col < V, final + g, -jnp.inf)
    m = jnp.max(y, axis=1)
    cand = jnp.where(y == m[:, None], col, jnp.int32(2**31 - 1))
    idx = jnp.min(cand, axis=1)

    @pl.when(j == 0)
    def _():
        m_scr[...] = m[:, None]
        i_scr[...] = idx[:, None]

    @pl.when(j > 0)
    def _():
        better = m[:, None] > m_scr[...]
        m_scr[...] = jnp.where(better, m[:, None], m_scr[...])
        i_scr[...] = jnp.where(better, idx[:, None], i_scr[...])

    @pl.when(j == NBLK - 1)
    def _():
        ids_ref[...] = i_scr[...]


def kernel(logits, prediction_mask):
    last = logits[:, -1, :]
    mask2 = prediction_mask[None, :]
    final, ids2d = pl.pallas_call(
        _compose_kernel,
        grid=(NBLK,),
        in_specs=[
            pl.BlockSpec((B, VB), lambda j: (0, j)),
            pl.BlockSpec((1, VB), lambda j: (0, j)),
        ],
        out_specs=[
            pl.BlockSpec((B, VB), lambda j: (0, j)),
            pl.BlockSpec((B, 1), lambda j: (0, 0)),
        ],
        out_shape=[
            jax.ShapeDtypeStruct((B, V), jnp.float32),
            jax.ShapeDtypeStruct((B, 1), jnp.int32),
        ],
        scratch_shapes=[
            pltpu.VMEM((B, 1), jnp.float32),
            pltpu.VMEM((B, 1), jnp.int32),
        ],
    )(last, mask2)
    return ids2d[:, 0], final
